# baseline (speedup 1.0000x reference)
import math

import jax
import jax.numpy as jnp
from jax import lax
from jax.experimental import pallas as pl
from jax.experimental.pallas import tpu as pltpu
from jax.experimental.pallas import tpu_sc as plsc

SOFTMAX_DIM = 32000
PADDING_TOKEN = 0
SMOOTHING_FACTOR = 0.1
CONFIDENCE = 1.0 - SMOOTHING_FACTOR
REDIST = SMOOTHING_FACTOR / (SOFTMAX_DIM - 2)
N_TOKENS = 2048
K_CONST = (SOFTMAX_DIM - 2) * REDIST * math.log(REDIST) + CONFIDENCE * math.log(CONFIDENCE)

LANES = 128
NW = 32
TOK_PER_W = N_TOKENS // NW

R_BLK = 32
G_BLK = N_TOKENS // R_BLK


def _sc_gather_kernel(x_hbm, tgt_hbm, out_hbm, tgt_v, rows_v, acc_v, sem):
    wid = lax.axis_index("s") * 2 + lax.axis_index("c")
    base = wid * TOK_PER_W
    pltpu.sync_copy(tgt_hbm.at[pl.ds(base, TOK_PER_W)], tgt_v)

    for jo in range(TOK_PER_W // 16):
        tv = tgt_v[pl.ds(jo * 16, 16)]
        cbv = tv - lax.bitwise_and(tv, 127)
        for ji in range(16):
            j = jo * 16 + ji
            tile_row = base + (j // 8) * 8
            cb = pl.multiple_of(cbv[ji], LANES)
            pltpu.async_copy(
                x_hbm.at[pl.ds(tile_row, 8), pl.ds(cb, LANES)],
                rows_v.at[j],
                sem,
            )
    for j in range(TOK_PER_W):
        pltpu.make_async_copy(
            x_hbm.at[pl.ds(0, 8), pl.ds(0, LANES)], rows_v.at[j], sem
        ).wait()

    acc = jnp.zeros((16,), jnp.float32)
    for j in range(TOK_PER_W // 16):
        t = tgt_v[pl.ds(j * 16, 16)]
        lane = lax.bitwise_and(t, 127)
        row_local = (j * 16) + lax.iota(jnp.int32, 16)
        sub_row = lax.bitwise_and(lax.iota(jnp.int32, 16), 7)
        gv = plsc.load_gather(rows_v, [row_local, sub_row, lane])
        m = jnp.where(t != PADDING_TOKEN, 1.0, 0.0).astype(jnp.float32)
        acc = acc + m * gv
    acc_v[...] = acc
    pltpu.sync_copy(acc_v, out_hbm.at[pl.ds(wid * 16, 16)])


def _sc_gather(x, tgt):
    mesh = plsc.VectorSubcoreMesh(core_axis_name="c", subcore_axis_name="s")
    return pl.kernel(
        _sc_gather_kernel,
        mesh=mesh,
        compiler_params=pltpu.CompilerParams(
            needs_layout_passes=False, use_tc_tiling_on_sc=True
        ),
        out_type=jax.ShapeDtypeStruct((NW * 16,), jnp.float32),
        scratch_types=[
            pltpu.VMEM((TOK_PER_W,), jnp.int32),
            pltpu.VMEM((TOK_PER_W, 8, LANES), jnp.float32),
            pltpu.VMEM((16,), jnp.float32),
            pltpu.SemaphoreType.DMA,
        ],
    )(x, tgt)


def _tc_body(x_ref, tgt_ref, out_ref):
    i = pl.program_id(0)
    x = x_ref[...]
    rs = jnp.sum(x, axis=1)
    x0 = x[:, 0]
    t = tgt_ref[0, 0, :]
    partial = jnp.sum(
        jnp.where(t != PADDING_TOKEN, K_CONST + REDIST * (x0 - rs), 0.0)
    )

    @pl.when(i == 0)
    def _init():
        out_ref[0, 0] = 0.0

    out_ref[0, 0] += partial


def _tc_part(x, tgt3):
    return pl.pallas_call(
        _tc_body,
        grid=(G_BLK,),
        in_specs=[
            pl.BlockSpec((R_BLK, SOFTMAX_DIM), lambda i: (i, 0)),
            pl.BlockSpec((1, 1, R_BLK), lambda i: (i, 0, 0)),
        ],
        out_specs=pl.BlockSpec(memory_space=pltpu.SMEM),
        out_shape=jax.ShapeDtypeStruct((1, 1), jnp.float32),
    )(x, tgt3)


def kernel(x, tgt_tokens):
    tgt = tgt_tokens.astype(jnp.int32)
    sg = _sc_gather(x, tgt)
    tgt3 = tgt.reshape(G_BLK, 1, R_BLK)
    tc_part = _tc_part(x, tgt3)
    return tc_part[0, 0] - (CONFIDENCE - REDIST) * jnp.sum(sg)


# speedup vs baseline: 1.1682x; 1.1682x over previous
import math

import jax
import jax.numpy as jnp
from jax import lax
from jax.experimental import pallas as pl
from jax.experimental.pallas import tpu as pltpu
from jax.experimental.pallas import tpu_sc as plsc

SOFTMAX_DIM = 32000
PADDING_TOKEN = 0
SMOOTHING_FACTOR = 0.1
CONFIDENCE = 1.0 - SMOOTHING_FACTOR
REDIST = SMOOTHING_FACTOR / (SOFTMAX_DIM - 2)
N_TOKENS = 2048
K_CONST = (SOFTMAX_DIM - 2) * REDIST * math.log(REDIST) + CONFIDENCE * math.log(CONFIDENCE)

LANES = 128
NW = 32
TOK_PER_W = N_TOKENS // NW

R_BLK = 64
G_BLK = N_TOKENS // R_BLK


def _sc_gather_kernel(x_hbm, tgt_hbm, out_hbm, tgt_v, rows_v, acc_v, sem):
    wid = lax.axis_index("s") * 2 + lax.axis_index("c")
    base = wid * TOK_PER_W
    pltpu.sync_copy(tgt_hbm.at[pl.ds(base, TOK_PER_W)], tgt_v)

    for jo in range(TOK_PER_W // 16):
        tv = tgt_v[pl.ds(jo * 16, 16)]
        cbv = tv - lax.bitwise_and(tv, 127)
        for ji in range(16):
            j = jo * 16 + ji
            tile_row = base + (j // 8) * 8
            cb = pl.multiple_of(cbv[ji], LANES)
            pltpu.async_copy(
                x_hbm.at[pl.ds(tile_row, 8), pl.ds(cb, LANES)],
                rows_v.at[j],
                sem,
            )
    for j in range(TOK_PER_W):
        pltpu.make_async_copy(
            x_hbm.at[pl.ds(0, 8), pl.ds(0, LANES)], rows_v.at[j], sem
        ).wait()

    acc = jnp.zeros((16,), jnp.float32)
    for j in range(TOK_PER_W // 16):
        t = tgt_v[pl.ds(j * 16, 16)]
        lane = lax.bitwise_and(t, 127)
        row_local = (j * 16) + lax.iota(jnp.int32, 16)
        sub_row = lax.bitwise_and(lax.iota(jnp.int32, 16), 7)
        gv = plsc.load_gather(rows_v, [row_local, sub_row, lane])
        m = jnp.where(t != PADDING_TOKEN, 1.0, 0.0).astype(jnp.float32)
        acc = acc + m * gv
    acc_v[...] = acc
    pltpu.sync_copy(acc_v, out_hbm.at[pl.ds(wid * 16, 16)])


def _sc_gather(x, tgt):
    mesh = plsc.VectorSubcoreMesh(core_axis_name="c", subcore_axis_name="s")
    return pl.kernel(
        _sc_gather_kernel,
        mesh=mesh,
        compiler_params=pltpu.CompilerParams(
            needs_layout_passes=False, use_tc_tiling_on_sc=True
        ),
        out_type=jax.ShapeDtypeStruct((NW * 16,), jnp.float32),
        scratch_types=[
            pltpu.VMEM((TOK_PER_W,), jnp.int32),
            pltpu.VMEM((TOK_PER_W, 8, LANES), jnp.float32),
            pltpu.VMEM((16,), jnp.float32),
            pltpu.SemaphoreType.DMA,
        ],
    )(x, tgt)


def _tc_body(x_ref, tgt_ref, out_ref):
    i = pl.program_id(0)
    x = x_ref[...]
    rs = jnp.sum(x, axis=1)
    x0 = x[:, 0]
    t = tgt_ref[0, 0, :]
    partial = jnp.sum(
        jnp.where(t != PADDING_TOKEN, K_CONST + REDIST * (x0 - rs), 0.0)
    )

    @pl.when(i == 0)
    def _init():
        out_ref[0, 0] = 0.0

    out_ref[0, 0] += partial


def _tc_part(x, tgt3):
    return pl.pallas_call(
        _tc_body,
        grid=(G_BLK,),
        in_specs=[
            pl.BlockSpec((R_BLK, SOFTMAX_DIM), lambda i: (i, 0)),
            pl.BlockSpec((1, 1, R_BLK), lambda i: (i, 0, 0)),
        ],
        out_specs=pl.BlockSpec(memory_space=pltpu.SMEM),
        out_shape=jax.ShapeDtypeStruct((1, 1), jnp.float32),
    )(x, tgt3)


def kernel(x, tgt_tokens):
    tgt = tgt_tokens.astype(jnp.int32)
    sg = _sc_gather(x, tgt)
    tgt3 = tgt.reshape(G_BLK, 1, R_BLK)
    tc_part = _tc_part(x, tgt3)
    return tc_part[0, 0] - (CONFIDENCE - REDIST) * jnp.sum(sg)
